# baseline (speedup 1.0000x reference)
import functools

import jax
import jax.numpy as jnp
from jax import lax
from jax.experimental import pallas as pl
from jax.experimental.pallas import tpu as pltpu
from jax.experimental.pallas import tpu_sc as plsc

_RX = _RY = _RZ = 72
_N, _C, _H, _W = 16, 3, 224, 224
_HW = _H * _W
_V = _RX * _RY * _RZ
_NW = 32
_PW = _N * _HW // _NW
_P = 512
_NCHUNK = _PW // _P
_G = _P // 16
_NT = _N * _V

_POFFS = (0, _RZ, _RY * _RZ, _RY * _RZ + _RZ)

_mesh = plsc.VectorSubcoreMesh(core_axis_name="c", subcore_axis_name="s")


@functools.partial(
    pl.kernel,
    out_type=jax.ShapeDtypeStruct((_N * _C, _HW), jnp.float32),
    mesh=_mesh,
    scratch_types=[
        [pltpu.VMEM((_C, _P), jnp.float32) for _ in range(2)],
        [pltpu.VMEM((4, _P), jnp.int32) for _ in range(2)],
        [pltpu.VMEM((_C, _P), jnp.float32) for _ in range(2)],
        [[pltpu.VMEM((_P, 8), jnp.float32) for _ in range(4)]
         for _ in range(2)],
        pltpu.VMEM((_C, _P), jnp.float32),
        [pltpu.SemaphoreType.DMA for _ in range(2)],
        pltpu.SemaphoreType.DMA,
    ],
    compiler_params=pltpu.CompilerParams(use_tc_tiling_on_sc=False,
                                         needs_layout_passes=False),
)
def _lut_kernel(imgs_hbm, lut_hbm, out_hbm, inbufs, idxbufs, fbufs, gbufs,
                obuf, gsems, isem):
    wid = lax.axis_index("s") * 2 + lax.axis_index("c")
    img = wid // 2
    base0 = (wid % 2) * _PW
    lut_base = img * _V
    iota = lax.iota(jnp.int32, 16)

    def fire(ci, b):
        base = base0 + ci * _P
        inbuf, idxbuf, fbuf = inbufs[b], idxbufs[b], fbufs[b]
        cps = [pltpu.async_copy(imgs_hbm.at[img * _C + c, pl.ds(base, _P)],
                                inbuf.at[c], isem) for c in range(_C)]
        for cp in cps:
            cp.wait()
        for g in range(_G):
            sl = pl.ds(g * 16, 16)
            sr = inbuf[0, sl] * float(_RX - 1)
            sg = inbuf[1, sl] * float(_RY - 1)
            sb = inbuf[2, sl] * float(_RZ - 1)
            ir = jnp.minimum(sr.astype(jnp.int32), _RX - 2)
            ig = jnp.minimum(sg.astype(jnp.int32), _RY - 2)
            ib = jnp.minimum(sb.astype(jnp.int32), _RZ - 2)
            fbuf[0, sl] = sr - ir.astype(jnp.float32)
            fbuf[1, sl] = sg - ig.astype(jnp.float32)
            fbuf[2, sl] = sb - ib.astype(jnp.float32)
            idx0 = (ir * _RY + ig) * _RZ + ib + lut_base
            for k, off in enumerate(_POFFS):
                idxbuf[k, sl] = idx0 + off
        for k in range(4):
            pltpu.async_copy(lut_hbm.at[idxbuf.at[k]], gbufs[b][k], gsems[b])

    def drain_blend(ci, b):
        base = base0 + ci * _P
        for k in range(4):
            pltpu.make_async_copy(lut_hbm.at[idxbufs[b].at[k]], gbufs[b][k],
                                  gsems[b]).wait()
        fbuf, gbuf = fbufs[b], gbufs[b]
        for g in range(_G):
            sl = pl.ds(g * 16, 16)
            fx = fbuf[0, sl]
            fy = fbuf[1, sl]
            fz = fbuf[2, sl]
            wz = [1.0 - fz, fz]
            wp = [(1.0 - fx) * (1.0 - fy), (1.0 - fx) * fy,
                  fx * (1.0 - fy), fx * fy]
            pix = iota + g * 16
            for c in range(_C):
                acc = jnp.zeros((16,), jnp.float32)
                for k in range(4):
                    for dz in range(2):
                        col = jnp.full((16,), 3 * dz + c, jnp.int32)
                        v = plsc.load_gather(gbuf[k], [pix, col])
                        acc = acc + (wp[k] * wz[dz]) * v
                obuf[c, sl] = jnp.clip(acc, 0.0, 1.0)

        for c in range(_C):
            pltpu.sync_copy(obuf.at[c],
                            out_hbm.at[img * _C + c, pl.ds(base, _P)])

    fire(0, 0)

    @pl.loop(0, _NCHUNK, step=2)
    def _chunk(ci):
        @pl.when(ci + 1 < _NCHUNK)
        def _():
            fire(ci + 1, 1)

        drain_blend(ci, 0)

        @pl.when(ci + 2 < _NCHUNK)
        def _():
            fire(ci + 2, 0)

        @pl.when(ci + 1 < _NCHUNK)
        def _():
            drain_blend(ci + 1, 1)


def kernel(imgs, xform_params):
    imgs_f = imgs.reshape(_N * _C, _HW)
    flat = xform_params.reshape(_NT * _C)
    flatp = jnp.concatenate([flat, jnp.zeros((8,), jnp.float32)])
    cols = [lax.slice(flatp, (j,), (j + _C * _NT,), (_C,)) for j in range(6)]
    zero = jnp.zeros((_NT,), jnp.float32)
    lut = jnp.stack(cols + [zero, zero], axis=1)
    out = _lut_kernel(imgs_f, lut)
    return out.reshape(_N, _C, _H, _W)


# speedup vs baseline: 5.9049x; 5.9049x over previous
import functools

import jax
import jax.numpy as jnp
from jax import lax
from jax.experimental import pallas as pl
from jax.experimental.pallas import tpu as pltpu
from jax.experimental.pallas import tpu_sc as plsc

_RX = _RY = _RZ = 72
_N, _C, _H, _W = 16, 3, 224, 224
_HW = _H * _W
_V = _RX * _RY * _RZ
_NW = 32
_PW = _N * _HW // _NW
_P = 512
_NCHUNK = _PW // _P
_G = _P // 16
_NT = _N * _V

_POFFS = (0, _RZ, _RY * _RZ, _RY * _RZ + _RZ)

_mesh = plsc.VectorSubcoreMesh(core_axis_name="c", subcore_axis_name="s")


@functools.partial(
    pl.kernel,
    out_type=jax.ShapeDtypeStruct((_N * _C, _HW), jnp.float32),
    mesh=_mesh,
    scratch_types=[
        [pltpu.VMEM((_C, _P), jnp.float32) for _ in range(2)],
        [pltpu.VMEM((4, _P), jnp.int32) for _ in range(2)],
        [pltpu.VMEM((_C, _P), jnp.float32) for _ in range(2)],
        [[pltpu.VMEM((_P, 8), jnp.float32) for _ in range(4)]
         for _ in range(2)],
        pltpu.VMEM((_C, _P), jnp.float32),
        [pltpu.SemaphoreType.DMA for _ in range(2)],
        pltpu.SemaphoreType.DMA,
    ],
    compiler_params=pltpu.CompilerParams(use_tc_tiling_on_sc=False,
                                         needs_layout_passes=False),
)
def _lut_kernel(imgs_hbm, lut_hbm, out_hbm, inbufs, idxbufs, fbufs, gbufs,
                obuf, gsems, isem):
    wid = lax.axis_index("s") * 2 + lax.axis_index("c")
    img = wid // 2
    base0 = (wid % 2) * _PW
    lut_base = img * _V
    iota = lax.iota(jnp.int32, 16)

    def fire(ci, b):
        base = base0 + ci * _P
        inbuf, idxbuf, fbuf = inbufs[b], idxbufs[b], fbufs[b]
        cps = [pltpu.async_copy(imgs_hbm.at[img * _C + c, pl.ds(base, _P)],
                                inbuf.at[c], isem) for c in range(_C)]
        for cp in cps:
            cp.wait()
        for g in range(_G):
            sl = pl.ds(g * 16, 16)
            sr = inbuf[0, sl] * float(_RX - 1)
            sg = inbuf[1, sl] * float(_RY - 1)
            sb = inbuf[2, sl] * float(_RZ - 1)
            ir = jnp.minimum(sr.astype(jnp.int32), _RX - 2)
            ig = jnp.minimum(sg.astype(jnp.int32), _RY - 2)
            ib = jnp.minimum(sb.astype(jnp.int32), _RZ - 2)
            fbuf[0, sl] = sr - ir.astype(jnp.float32)
            fbuf[1, sl] = sg - ig.astype(jnp.float32)
            fbuf[2, sl] = sb - ib.astype(jnp.float32)
            idx0 = (ir * _RY + ig) * _RZ + ib + lut_base
            for k, off in enumerate(_POFFS):
                idxbuf[k, sl] = idx0 + off
        for k in range(4):
            pltpu.async_copy(lut_hbm.at[idxbuf.at[k]], gbufs[b][k], gsems[b])

    def drain_blend(ci, b):
        base = base0 + ci * _P
        for k in range(4):
            pltpu.make_async_copy(lut_hbm.at[idxbufs[b].at[k]], gbufs[b][k],
                                  gsems[b]).wait()
        fbuf, gbuf = fbufs[b], gbufs[b]
        for g in range(_G):
            sl = pl.ds(g * 16, 16)
            fx = fbuf[0, sl]
            fy = fbuf[1, sl]
            fz = fbuf[2, sl]
            wz = [1.0 - fz, fz]
            wp = [(1.0 - fx) * (1.0 - fy), (1.0 - fx) * fy,
                  fx * (1.0 - fy), fx * fy]
            pix = iota + g * 16
            for c in range(_C):
                acc = jnp.zeros((16,), jnp.float32)
                for k in range(4):
                    for dz in range(2):
                        col = jnp.full((16,), 3 * dz + c, jnp.int32)
                        v = plsc.load_gather(gbuf[k], [pix, col])
                        acc = acc + (wp[k] * wz[dz]) * v
                obuf[c, sl] = jnp.clip(acc, 0.0, 1.0)

        for c in range(_C):
            pltpu.sync_copy(obuf.at[c],
                            out_hbm.at[img * _C + c, pl.ds(base, _P)])

    fire(0, 0)

    @pl.loop(0, _NCHUNK, step=2)
    def _chunk(ci):
        @pl.when(ci + 1 < _NCHUNK)
        def _():
            fire(ci + 1, 1)

        drain_blend(ci, 0)

        @pl.when(ci + 2 < _NCHUNK)
        def _():
            fire(ci + 2, 0)

        @pl.when(ci + 1 < _NCHUNK)
        def _():
            drain_blend(ci + 1, 1)


def kernel(imgs, xform_params):
    imgs_f = imgs.reshape(_N * _C, _HW)
    rows = xform_params.reshape(_NT, _C)
    rows1 = jnp.concatenate([rows[1:], rows[:1]], axis=0)
    pad2 = jnp.zeros((_NT, 2), jnp.float32)
    lut = jnp.concatenate([rows, rows1, pad2], axis=1)
    out = _lut_kernel(imgs_f, lut)
    return out.reshape(_N, _C, _H, _W)
